# baseline (speedup 1.0000x reference)
import numpy as np
import jax
import jax.numpy as jnp
from jax.experimental import pallas as pl
from jax.experimental.pallas import tpu as pltpu

_A = 3
_ATTR = 85
_G = 76
_S = _G * _G
_STRIDE = 8.0
_ANCH_W = (116.0, 156.0, 373.0)
_ANCH_H = (90.0, 198.0, 326.0)


def _make_addm():
    p = np.arange(_S)
    addm = np.zeros((_A * _S, _ATTR), dtype=np.float32)
    for a in range(_A):
        addm[a * _S:(a + 1) * _S, 0] = (p % _G) * _STRIDE
        addm[a * _S:(a + 1) * _S, 1] = (p // _G) * _STRIDE
    return addm


def _make_mult():
    m = np.ones((_A, 1, _ATTR), dtype=np.float32)
    m[:, 0, 0:2] = _STRIDE
    for a in range(_A):
        m[a, 0, 2] = _ANCH_W[a]
        m[a, 0, 3] = _ANCH_H[a]
    return m


_ADDM = _make_addm()
_MULT = _make_mult()


def _decode(x_ref, addm_ref, mult_ref, o_ref):
    li = jax.lax.broadcasted_iota(jnp.int32, (1, _ATTR), 1)
    is_wh = (li == 2) | (li == 3)

    x = x_ref[0]
    for a in range(_A):
        xa = x[:, a * _ATTR:(a + 1) * _ATTR]
        sig = 0.5 * jnp.tanh(0.5 * xa) + 0.5
        val = jnp.where(is_wh, jnp.exp(xa), sig)
        o_ref[0, a * _S:(a + 1) * _S, :] = (
            val * mult_ref[a] + addm_ref[a * _S:(a + 1) * _S, :]
        )


def _decode_alias(x_ref, addm_ref, mult_ref, prev_ref, o_ref):
    del prev_ref
    _decode(x_ref, addm_ref, mult_ref, o_ref)


def kernel(inputs):
    b = inputs.shape[0]
    h = b // 2
    x3 = inputs.reshape(b, _A * _ATTR, _S)
    xt0 = jnp.transpose(x3[:h], (0, 2, 1))
    xt1 = jnp.transpose(x3[h:], (0, 2, 1))

    in_specs = [
        pl.BlockSpec((1, _S, _A * _ATTR), lambda i: (i, 0, 0)),
        pl.BlockSpec((_A * _S, _ATTR), lambda i: (0, 0)),
        pl.BlockSpec((_A, 1, _ATTR), lambda i: (0, 0, 0)),
    ]
    out_shape = jax.ShapeDtypeStruct((b, _A * _S, _ATTR), jnp.float32)

    out0 = pl.pallas_call(
        _decode,
        grid=(h,),
        in_specs=in_specs,
        out_specs=pl.BlockSpec((1, _A * _S, _ATTR), lambda i: (i, 0, 0)),
        out_shape=out_shape,
    )(xt0, _ADDM, _MULT)

    return pl.pallas_call(
        _decode_alias,
        grid=(h,),
        in_specs=in_specs + [pl.BlockSpec(memory_space=pl.ANY)],
        out_specs=pl.BlockSpec((1, _A * _S, _ATTR), lambda i, _h=h: (i + _h, 0, 0)),
        out_shape=out_shape,
        input_output_aliases={3: 0},
    )(xt1, _ADDM, _MULT, out0)


# speedup vs baseline: 1.1975x; 1.1975x over previous
import numpy as np
import jax
import jax.numpy as jnp
from jax.experimental import pallas as pl

_A = 3
_ATTR = 85
_G = 76
_S = _G * _G
_STRIDE = 8.0
_ANCH_W = (116.0, 156.0, 373.0)
_ANCH_H = (90.0, 198.0, 326.0)


def _make_addm():
    p = np.arange(_S)
    addm = np.zeros((_A * _S, _ATTR), dtype=np.float32)
    for a in range(_A):
        addm[a * _S:(a + 1) * _S, 0] = (p % _G) * _STRIDE
        addm[a * _S:(a + 1) * _S, 1] = (p // _G) * _STRIDE
    return addm


def _make_mult():
    m = np.ones((_A, 1, _ATTR), dtype=np.float32)
    m[:, 0, 0:2] = _STRIDE
    for a in range(_A):
        m[a, 0, 2] = _ANCH_W[a]
        m[a, 0, 3] = _ANCH_H[a]
    return m


_ADDM = _make_addm()
_MULT = _make_mult()


def _decode_kernel(x_ref, addm_ref, mult_ref, o_ref):
    li = jax.lax.broadcasted_iota(jnp.int32, (1, _ATTR), 1)
    is_wh = (li == 2) | (li == 3)

    x = x_ref[0]
    for a in range(_A):
        xa = x[:, a * _ATTR:(a + 1) * _ATTR]
        sig = 0.5 * jnp.tanh(0.5 * xa) + 0.5
        val = jnp.where(is_wh, jnp.exp(xa), sig)
        o_ref[0, a * _S:(a + 1) * _S, :] = (
            val * mult_ref[a] + addm_ref[a * _S:(a + 1) * _S, :]
        )


def kernel(inputs):
    b = inputs.shape[0]
    xt = jnp.transpose(inputs.reshape(b, _A * _ATTR, _S), (0, 2, 1))
    return pl.pallas_call(
        _decode_kernel,
        grid=(b,),
        in_specs=[
            pl.BlockSpec((1, _S, _A * _ATTR), lambda i: (i, 0, 0)),
            pl.BlockSpec((_A * _S, _ATTR), lambda i: (0, 0)),
            pl.BlockSpec((_A, 1, _ATTR), lambda i: (0, 0, 0)),
        ],
        out_specs=pl.BlockSpec((1, _A * _S, _ATTR), lambda i: (i, 0, 0)),
        out_shape=jax.ShapeDtypeStruct((b, _A * _S, _ATTR), jnp.float32),
    )(xt, _ADDM, _MULT)
